# baseline (speedup 1.0000x reference)
import functools

import jax
import jax.numpy as jnp
from jax import lax
from jax.experimental import pallas as pl
from jax.experimental.pallas import tpu as pltpu
from jax.experimental.pallas import tpu_sc as plsc

HIDDEN = 64
BATCH = 16384
NC = 2
NS = 16
L = 16
NW = NC * NS
BPW = BATCH // NW
ICH = 128
NCH = BPW // ICH

HALF_PI = 1.5707963267948966
INV_2PI = 0.15915494309189535
C1 = 6.28125
C2 = 2 * 3.141592653589793 - C1

S = (9.99984590e-01, -1.66632589e-01, 8.31238590e-03,
     -1.93162309e-04, 2.17323611e-06)


def _sin_poly(x):
    q = x * INV_2PI
    h = jnp.where(q >= 0.0, 0.5, -0.5).astype(jnp.float32)
    k = (q + h).astype(jnp.int32).astype(jnp.float32)
    r = x - k * C1
    r = r - k * C2
    z = r * r
    p = jnp.float32(S[4])
    p = p * z + S[3]
    p = p * z + S[2]
    p = p * z + S[1]
    p = p * z + S[0]
    return p * r


def _body(t_hbm, w_hbm, para_hbm, phase_hbm, out_hbm,
          idx_v, t_v, amp_v, ph_v, sem):
    wid = lax.axis_index("s") * NC + lax.axis_index("c")
    base = wid * BPW

    pltpu.sync_copy(w_hbm.at[wid], idx_v)
    copies = []
    for j in range(NCH):
        copies.append(pltpu.async_copy(
            para_hbm.at[idx_v.at[j]], amp_v.at[pl.ds(j * ICH, ICH)], sem))
        copies.append(pltpu.async_copy(
            phase_hbm.at[idx_v.at[j]], ph_v.at[pl.ds(j * ICH, ICH)], sem))
    pltpu.sync_copy(t_hbm.at[wid], t_v)
    for c in copies:
        c.wait()

    def row(i, carry):
        t = t_v[i, :]
        for j in range(HIDDEN // L):
            x = amp_v[i, pl.ds(j * L, L)] * t + ph_v[i, pl.ds(j * L, L)]
            if j < (HIDDEN // L) // 2:
                x = x + HALF_PI
            amp_v[i, pl.ds(j * L, L)] = _sin_poly(x)
        return carry

    lax.fori_loop(0, BPW, row, 0)
    pltpu.sync_copy(amp_v, out_hbm.at[pl.ds(base, BPW)])


@functools.partial(jax.jit, static_argnames=())
def kernel(_time, word, para_table, phase_table):
    t16 = jnp.broadcast_to(_time[:, None], (BATCH, L)).reshape(NW, BPW, L)
    w = word.astype(jnp.int32).reshape(NW, NCH, ICH)
    mesh = plsc.VectorSubcoreMesh(core_axis_name="c", subcore_axis_name="s")
    f = pl.kernel(
        _body,
        mesh=mesh,
        out_type=jax.ShapeDtypeStruct((BATCH, HIDDEN), jnp.float32),
        scratch_types=[
            pltpu.VMEM((NCH, ICH), jnp.int32),
            pltpu.VMEM((BPW, L), jnp.float32),
            pltpu.VMEM((BPW, HIDDEN), jnp.float32),
            pltpu.VMEM((BPW, HIDDEN), jnp.float32),
            pltpu.SemaphoreType.DMA,
        ],
        compiler_params=pltpu.CompilerParams(use_tc_tiling_on_sc=False),
    )
    return f(t16, w, para_table, phase_table)
